# baseline (speedup 1.0000x reference)
import jax
import jax.numpy as jnp
from jax import lax
from jax.experimental import pallas as pl
from jax.experimental.pallas import tpu as pltpu

B = 2
NUM_POINTS = 512
FEAT_DIM = 32
RES = 128
RAY_SUB = 112
DEPTH_RES = 128
K = 8
NFREQ = 10
NEAR = 0.5
FAR = 4.5

R_BLK = 16
S_BLK = R_BLK * DEPTH_RES
N_BLK = RAY_SUB // R_BLK
ENC2 = 2 * NFREQ

_HI = lax.Precision.HIGHEST
_BF = jnp.bfloat16


def _leaky(x):
    return jnp.where(x >= 0, x, 0.01 * x)


def _dot(a, b):
    return lax.dot_general(a, b, (((1,), (0,)), ((), ())), precision=_HI)


def _bdot(a, b):
    return lax.dot_general(a.astype(_BF), b, (((1,), (0,)), ((), ())),
                           preferred_element_type=jnp.float32)


def _split(a):
    hi = a.astype(_BF)
    lo = (a - hi.astype(jnp.float32)).astype(_BF)
    return hi, lo


def _hldot(a, b_bf):
    hi, lo = _split(a)
    return _bdot(hi, b_bf) + _bdot(lo, b_bf)


def _bbdot(a, b):
    return lax.dot_general(a, b, (((1,), (0,)), ((), ())),
                           preferred_element_type=jnp.float32)


def _pointnerf_block(p_ref, c_ref, g_ref,
                     w0_ref, b0_ref,
                     w1_ref, b1_ref, w2_ref, b2_ref, w3_ref, b3_ref,
                     w4_ref, b4_ref,
                     f0_ref, fb0_ref, f1_ref, fb1_ref,
                     f2_ref, fb2_ref, f3_ref, fb3_ref,
                     s0_ref, sb0_ref, s1_ref, sb1_ref,
                     rw_ref, rb_ref,
                     out_ref):
    p = p_ref[0]
    coords = c_ref[0]
    g = g_ref[0]

    pp = jnp.sum(p * p, axis=1, keepdims=True)
    cc = jnp.sum(coords * coords, axis=1)[None, :]
    ph, pl_ = _split(p)
    ch = g[:, 32:35]
    cl = g[:, 35:38]

    def dott(a, b):
        return lax.dot_general(a, b, (((1,), (1,)), ((), ())),
                               preferred_element_type=jnp.float32)

    pc = dott(ph, ch) + dott(ph, cl) + dott(pl_, ch)
    d2 = pp + cc - 2.0 * pc

    iota = lax.broadcasted_iota(jnp.int32, d2.shape, 1)
    key = (lax.bitcast_convert_type(jnp.maximum(d2, 0.0), jnp.int32)
           & jnp.int32(~511)) | iota
    idxs = []
    dists = []
    for _ in range(K):
        kmin = jnp.min(key, axis=1, keepdims=True)
        key = jnp.where(key == kmin, jnp.int32(0x7FFFFFFF), key)
        idxs.append(kmin & 511)
        dists.append(lax.bitcast_convert_type(kmin & jnp.int32(~511),
                                              jnp.float32))
    nn_d2 = jnp.concatenate(dists, axis=1)
    w = 1.0 / (nn_d2 + 1e-8)
    w = w / jnp.sum(w, axis=1, keepdims=True)

    jj = lax.broadcasted_iota(jnp.int32, (3, 3 * ENC2), 1)
    dd = lax.broadcasted_iota(jnp.int32, (3, 3 * ENC2), 0)
    sel_mat = jnp.where(jj // ENC2 == dd,
                        jnp.exp2((jj % NFREQ).astype(jnp.float32)),
                        0.0).astype(_BF)
    is_sin = (lax.broadcasted_iota(jnp.int32, (1, 3 * ENC2), 1) % ENC2) < NFREQ

    feat = jnp.zeros((S_BLK, 256), jnp.float32)
    for k in range(K):
        oh = (iota == idxs[k]).astype(_BF)
        nb = _bbdot(oh, g)
        nbc = nb[:, 32:35] + nb[:, 35:38]
        rel = p - nbc
        scaled = _hldot(rel, sel_mat)
        enc = jnp.where(is_sin, jnp.sin(scaled), jnp.cos(scaled))
        x = jnp.concatenate([nb[:, :32], rel, enc], axis=1)
        h = _leaky(_bdot(x, w0_ref[...]) + b0_ref[...])
        h = _leaky(_bdot(h, w1_ref[...]) + b1_ref[...])
        h = _leaky(_bdot(h, w2_ref[...]) + b2_ref[...])
        h = _leaky(_bdot(h, w3_ref[...]) + b3_ref[...])
        h = _bdot(h, w4_ref[...]) + b4_ref[...]
        feat = feat + w[:, k:k + 1] * h

    t = _leaky(_bdot(feat, f0_ref[...]) + fb0_ref[...])
    t = _leaky(_bdot(t, f1_ref[...]) + fb1_ref[...])
    t = _leaky(_bdot(t, f2_ref[...]) + fb2_ref[...])
    t = _leaky(_bdot(t, f3_ref[...]) + fb3_ref[...])

    hidden = _leaky(_bdot(t, s0_ref[...]) + sb0_ref[...])
    sig_pre = jnp.sum(hidden * s1_ref[...], axis=1, keepdims=True) \
        + sb1_ref[0:1, 0:1]
    sigma = jax.nn.softplus(sig_pre)
    rgb = jax.nn.sigmoid(_bdot(t, rw_ref[...]) + rb_ref[...])

    delta = (FAR - NEAR) / (DEPTH_RES - 1)
    et = jnp.exp(-sigma.reshape(R_BLK, DEPTH_RES) * delta)
    alpha = 1.0 - et
    la = jnp.log(et + 1e-10)
    ji = lax.broadcasted_iota(jnp.int32, (DEPTH_RES, DEPTH_RES), 0)
    ii = lax.broadcasted_iota(jnp.int32, (DEPTH_RES, DEPTH_RES), 1)
    tri = (ji < ii).astype(_BF)
    lah, lal = _split(la)
    trans = jnp.exp(_bbdot(lah, tri) + _bbdot(lal, tri))
    wts = alpha * trans
    acc = jnp.sum(wts, axis=1, keepdims=True)
    cr = [jnp.sum(wts * rgb[:, c:c + 1].reshape(R_BLK, DEPTH_RES), axis=1,
                  keepdims=True) for c in range(3)]
    out_ref[0] = jnp.concatenate(cr, axis=1) + (1.0 - acc)


def kernel(obj_idx, intrinsics, extrinsics, sample_rays, feats_table,
           coords_table, ray_idx, agg_W0, agg_b0, agg_W1, agg_b1, agg_W2,
           agg_b2, agg_W3, agg_b3, agg_W4, agg_b4, fld_W0, fld_b0, fld_W1,
           fld_b1, fld_W2, fld_b2, fld_W3, fld_b3, sig_W0, sig_b0, sig_W1,
           sig_b1, rgb_W, rgb_b):
    feats = feats_table[obj_idx][..., :FEAT_DIM]
    coords = coords_table[obj_idx]

    sel = jnp.where(sample_rays != 0, ray_idx,
                    jnp.arange(RAY_SUB, dtype=ray_idx.dtype))
    u = (sel % RES).astype(jnp.float32) + 0.5
    v = (sel // RES).astype(jnp.float32) + 0.5
    pix = jnp.stack([u, v, jnp.ones_like(u)], axis=-1)
    Kinv = jnp.linalg.inv(intrinsics)
    dirs_cam = jnp.einsum('bij,pj->bpi', Kinv, pix)
    dirs = jnp.einsum('bij,bpj->bpi', extrinsics[:, :3, :3], dirs_cam)
    dirs = dirs / (jnp.linalg.norm(dirs, axis=-1, keepdims=True) + 1e-8)
    origins = extrinsics[:, :3, 3]
    tvals = jnp.linspace(NEAR, FAR, DEPTH_RES, dtype=jnp.float32)
    pts = origins[:, None, None, :] + dirs[:, :, None, :] * tvals[None, None, :, None]
    p = pts.reshape(B, RAY_SUB * DEPTH_RES, 3)

    row = lambda b: b.reshape(1, -1)
    s1r = sig_W1.reshape(1, 256)
    sb1m = jnp.broadcast_to(sig_b1.reshape(1, 1), (1, 128))
    rb = rgb_b.reshape(1, 3)

    def const_spec(a):
        return pl.BlockSpec(a.shape, lambda b, i: (0,) * a.ndim)

    bf = lambda a: a.astype(_BF)
    ch = bf(coords)
    cl = bf(coords - ch.astype(jnp.float32))
    gcat = jnp.concatenate([bf(feats), ch, cl], axis=-1)
    weights = [bf(agg_W0), row(agg_b0),
               bf(agg_W1), row(agg_b1), bf(agg_W2), row(agg_b2),
               bf(agg_W3), row(agg_b3), bf(agg_W4), row(agg_b4),
               bf(fld_W0), row(fld_b0), bf(fld_W1), row(fld_b1),
               bf(fld_W2), row(fld_b2), bf(fld_W3), row(fld_b3),
               bf(sig_W0), row(sig_b0), s1r, sb1m, bf(rgb_W), rb]

    out = pl.pallas_call(
        _pointnerf_block,
        grid=(B, N_BLK),
        in_specs=[
            pl.BlockSpec((1, S_BLK, 3), lambda b, i: (b, i, 0)),
            pl.BlockSpec((1, NUM_POINTS, 3), lambda b, i: (b, 0, 0)),
            pl.BlockSpec((1, NUM_POINTS, 38), lambda b, i: (b, 0, 0)),
        ] + [const_spec(a) for a in weights],
        out_specs=pl.BlockSpec((1, R_BLK, 3), lambda b, i: (b, i, 0)),
        out_shape=jax.ShapeDtypeStruct((B, RAY_SUB, 3), jnp.float32),
        compiler_params=pltpu.CompilerParams(
            dimension_semantics=("parallel", "arbitrary")),
    )(p, coords, gcat, *weights)
    return out


# speedup vs baseline: 1.1020x; 1.1020x over previous
import jax
import jax.numpy as jnp
from jax import lax
from jax.experimental import pallas as pl
from jax.experimental.pallas import tpu as pltpu

B = 2
NUM_POINTS = 512
FEAT_DIM = 32
RES = 128
RAY_SUB = 112
DEPTH_RES = 128
K = 8
NFREQ = 10
NEAR = 0.5
FAR = 4.5

R_BLK = 8
S_BLK = R_BLK * DEPTH_RES
N_BLK = RAY_SUB // R_BLK
ENC2 = 2 * NFREQ

_HI = lax.Precision.HIGHEST
_BF = jnp.bfloat16


def _leaky(x):
    return jnp.where(x >= 0, x, 0.01 * x)


def _dot(a, b):
    return lax.dot_general(a, b, (((1,), (0,)), ((), ())), precision=_HI)


def _bdot(a, b):
    return lax.dot_general(a.astype(_BF), b, (((1,), (0,)), ((), ())),
                           preferred_element_type=jnp.float32)


def _split(a):
    hi = a.astype(_BF)
    lo = (a - hi.astype(jnp.float32)).astype(_BF)
    return hi, lo


def _hldot(a, b_bf):
    hi, lo = _split(a)
    return _bdot(hi, b_bf) + _bdot(lo, b_bf)


def _bbdot(a, b):
    return lax.dot_general(a, b, (((1,), (0,)), ((), ())),
                           preferred_element_type=jnp.float32)


def _pointnerf_block(p_ref, c_ref, g_ref,
                     w0_ref, b0_ref,
                     w1_ref, b1_ref, w2_ref, b2_ref, w3_ref, b3_ref,
                     w4_ref, b4_ref,
                     f0_ref, fb0_ref, f1_ref, fb1_ref,
                     f2_ref, fb2_ref, f3_ref, fb3_ref,
                     s0_ref, sb0_ref, s1_ref, sb1_ref,
                     rw_ref, rb_ref,
                     out_ref):
    p = p_ref[0]
    coords = c_ref[0]
    g = g_ref[0]

    pp = jnp.sum(p * p, axis=1, keepdims=True)
    cc = jnp.sum(coords * coords, axis=1)[None, :]
    ph, pl_ = _split(p)
    ch = g[:, 32:35]
    cl = g[:, 35:38]

    def dott(a, b):
        return lax.dot_general(a, b, (((1,), (1,)), ((), ())),
                               preferred_element_type=jnp.float32)

    pc = dott(ph, ch) + dott(ph, cl) + dott(pl_, ch)
    d2 = pp + cc - 2.0 * pc

    iota = lax.broadcasted_iota(jnp.int32, d2.shape, 1)
    key = (lax.bitcast_convert_type(jnp.maximum(d2, 0.0), jnp.int32)
           & jnp.int32(~511)) | iota
    idxs = []
    dists = []
    for _ in range(K):
        kmin = jnp.min(key, axis=1, keepdims=True)
        key = jnp.where(key == kmin, jnp.int32(0x7FFFFFFF), key)
        idxs.append(kmin & 511)
        dists.append(lax.bitcast_convert_type(kmin & jnp.int32(~511),
                                              jnp.float32))
    nn_d2 = jnp.concatenate(dists, axis=1)
    w = 1.0 / (nn_d2 + 1e-8)
    w = w / jnp.sum(w, axis=1, keepdims=True)

    jj = lax.broadcasted_iota(jnp.int32, (3, 3 * ENC2), 1)
    dd = lax.broadcasted_iota(jnp.int32, (3, 3 * ENC2), 0)
    sel_mat = jnp.where(jj // ENC2 == dd,
                        jnp.exp2((jj % NFREQ).astype(jnp.float32)),
                        0.0).astype(_BF)
    is_sin = (lax.broadcasted_iota(jnp.int32, (1, 3 * ENC2), 1) % ENC2) < NFREQ

    feat = jnp.zeros((S_BLK, 256), jnp.float32)
    for k in range(K):
        oh = (iota == idxs[k]).astype(_BF)
        nb = _bbdot(oh, g)
        nbc = nb[:, 32:35] + nb[:, 35:38]
        rel = p - nbc
        scaled = _hldot(rel, sel_mat)
        enc = jnp.where(is_sin, jnp.sin(scaled), jnp.cos(scaled))
        x = jnp.concatenate([nb[:, :32], rel, enc], axis=1)
        h = _leaky(_bdot(x, w0_ref[...]) + b0_ref[...])
        h = _leaky(_bdot(h, w1_ref[...]) + b1_ref[...])
        h = _leaky(_bdot(h, w2_ref[...]) + b2_ref[...])
        h = _leaky(_bdot(h, w3_ref[...]) + b3_ref[...])
        h = _bdot(h, w4_ref[...]) + b4_ref[...]
        feat = feat + w[:, k:k + 1] * h

    t = _leaky(_bdot(feat, f0_ref[...]) + fb0_ref[...])
    t = _leaky(_bdot(t, f1_ref[...]) + fb1_ref[...])
    t = _leaky(_bdot(t, f2_ref[...]) + fb2_ref[...])
    t = _leaky(_bdot(t, f3_ref[...]) + fb3_ref[...])

    hidden = _leaky(_bdot(t, s0_ref[...]) + sb0_ref[...])
    sig_pre = jnp.sum(hidden * s1_ref[...], axis=1, keepdims=True) \
        + sb1_ref[0:1, 0:1]
    sigma = jax.nn.softplus(sig_pre)
    rgb = jax.nn.sigmoid(_bdot(t, rw_ref[...]) + rb_ref[...])

    delta = (FAR - NEAR) / (DEPTH_RES - 1)
    et = jnp.exp(-sigma.reshape(R_BLK, DEPTH_RES) * delta)
    alpha = 1.0 - et
    la = jnp.log(et + 1e-10)
    ji = lax.broadcasted_iota(jnp.int32, (DEPTH_RES, DEPTH_RES), 0)
    ii = lax.broadcasted_iota(jnp.int32, (DEPTH_RES, DEPTH_RES), 1)
    tri = (ji < ii).astype(_BF)
    lah, lal = _split(la)
    trans = jnp.exp(_bbdot(lah, tri) + _bbdot(lal, tri))
    wts = alpha * trans
    acc = jnp.sum(wts, axis=1, keepdims=True)
    cr = [jnp.sum(wts * rgb[:, c:c + 1].reshape(R_BLK, DEPTH_RES), axis=1,
                  keepdims=True) for c in range(3)]
    out_ref[0] = jnp.concatenate(cr, axis=1) + (1.0 - acc)


def kernel(obj_idx, intrinsics, extrinsics, sample_rays, feats_table,
           coords_table, ray_idx, agg_W0, agg_b0, agg_W1, agg_b1, agg_W2,
           agg_b2, agg_W3, agg_b3, agg_W4, agg_b4, fld_W0, fld_b0, fld_W1,
           fld_b1, fld_W2, fld_b2, fld_W3, fld_b3, sig_W0, sig_b0, sig_W1,
           sig_b1, rgb_W, rgb_b):
    feats = feats_table[obj_idx][..., :FEAT_DIM]
    coords = coords_table[obj_idx]

    sel = jnp.where(sample_rays != 0, ray_idx,
                    jnp.arange(RAY_SUB, dtype=ray_idx.dtype))
    u = (sel % RES).astype(jnp.float32) + 0.5
    v = (sel // RES).astype(jnp.float32) + 0.5
    pix = jnp.stack([u, v, jnp.ones_like(u)], axis=-1)
    Kinv = jnp.linalg.inv(intrinsics)
    dirs_cam = jnp.einsum('bij,pj->bpi', Kinv, pix)
    dirs = jnp.einsum('bij,bpj->bpi', extrinsics[:, :3, :3], dirs_cam)
    dirs = dirs / (jnp.linalg.norm(dirs, axis=-1, keepdims=True) + 1e-8)
    origins = extrinsics[:, :3, 3]
    tvals = jnp.linspace(NEAR, FAR, DEPTH_RES, dtype=jnp.float32)
    pts = origins[:, None, None, :] + dirs[:, :, None, :] * tvals[None, None, :, None]
    p = pts.reshape(B, RAY_SUB * DEPTH_RES, 3)

    row = lambda b: b.reshape(1, -1)
    s1r = sig_W1.reshape(1, 256)
    sb1m = jnp.broadcast_to(sig_b1.reshape(1, 1), (1, 128))
    rb = rgb_b.reshape(1, 3)

    def const_spec(a):
        return pl.BlockSpec(a.shape, lambda b, i: (0,) * a.ndim)

    bf = lambda a: a.astype(_BF)
    ch = bf(coords)
    cl = bf(coords - ch.astype(jnp.float32))
    gcat = jnp.concatenate([bf(feats), ch, cl], axis=-1)
    weights = [bf(agg_W0), row(agg_b0),
               bf(agg_W1), row(agg_b1), bf(agg_W2), row(agg_b2),
               bf(agg_W3), row(agg_b3), bf(agg_W4), row(agg_b4),
               bf(fld_W0), row(fld_b0), bf(fld_W1), row(fld_b1),
               bf(fld_W2), row(fld_b2), bf(fld_W3), row(fld_b3),
               bf(sig_W0), row(sig_b0), s1r, sb1m, bf(rgb_W), rb]

    out = pl.pallas_call(
        _pointnerf_block,
        grid=(B, N_BLK),
        in_specs=[
            pl.BlockSpec((1, S_BLK, 3), lambda b, i: (b, i, 0)),
            pl.BlockSpec((1, NUM_POINTS, 3), lambda b, i: (b, 0, 0)),
            pl.BlockSpec((1, NUM_POINTS, 38), lambda b, i: (b, 0, 0)),
        ] + [const_spec(a) for a in weights],
        out_specs=pl.BlockSpec((1, R_BLK, 3), lambda b, i: (b, i, 0)),
        out_shape=jax.ShapeDtypeStruct((B, RAY_SUB, 3), jnp.float32),
        compiler_params=pltpu.CompilerParams(
            dimension_semantics=("parallel", "arbitrary")),
    )(p, coords, gcat, *weights)
    return out
